# baseline (speedup 1.0000x reference)
import functools

import jax
import jax.numpy as jnp
from jax import lax
from jax.experimental import pallas as pl
from jax.experimental.pallas import tpu as pltpu
from jax.experimental.pallas import tpu_sc as plsc

F32 = jnp.float32
I32 = jnp.int32

NC, NS, LANES = 2, 16, 16
NW = NC * NS
BN = 512
D = 40


def _mesh():
    return plsc.VectorSubcoreMesh(core_axis_name="c", subcore_axis_name="s")


def _zero_1d(ref, n):
    z16 = jnp.zeros((16,), F32)

    def body(i, c):
        ref[pl.ds(i * 16, 16)] = z16
        return c

    lax.fori_loop(0, n // 16, body, 0)


def _sc_degree(src_f, he_f, Np, Mp):
    Ep = src_f.shape[0]
    EPT = Ep // NW
    CH = 1600

    @functools.partial(
        pl.kernel,
        out_type=[
            jax.ShapeDtypeStruct((NW, Np), F32),
            jax.ShapeDtypeStruct((NW, Mp), F32),
        ],
        mesh=_mesh(),
        compiler_params=pltpu.CompilerParams(needs_layout_passes=False),
        scratch_types=[
            pltpu.VMEM((Np,), F32),
            pltpu.VMEM((Mp,), F32),
            pltpu.VMEM((CH,), I32),
            pltpu.VMEM((CH,), I32),
            pltpu.SemaphoreType.DMA,
        ],
    )
    def k(src_ref, he_ref, on_ref, om_ref, cn_v, cm_v, si_v, hi_v, semi):
        wid = lax.axis_index("s") * NC + lax.axis_index("c")
        _zero_1d(cn_v, Np)
        _zero_1d(cm_v, Mp)
        ones = jnp.ones((16,), F32)
        base0 = wid * EPT

        def chunk(c, carry):
            b = base0 + c * CH
            d1 = pltpu.async_copy(src_ref.at[pl.ds(b, CH)], si_v, semi)
            d2 = pltpu.async_copy(he_ref.at[pl.ds(b, CH)], hi_v, semi)
            d1.wait()
            d2.wait()

            @functools.partial(plsc.parallel_loop, 0, CH // 16, unroll=2)
            def grp(g):
                o = g * 16
                plsc.addupdate_scatter(cn_v, [si_v[pl.ds(o, 16)]], ones)
                plsc.addupdate_scatter(cm_v, [hi_v[pl.ds(o, 16)]], ones)

            return carry

        lax.fori_loop(0, EPT // CH, chunk, 0)
        pltpu.sync_copy(cn_v, on_ref.at[wid])
        pltpu.sync_copy(cm_v, om_ref.at[wid])

    return k(src_f, he_f)


def _sc_alpha(src_f, he_f, sx, se, mxs, mxe):
    Ep = src_f.shape[0]
    Np = sx.shape[0]
    Mp = se.shape[0]
    EPT = Ep // NW
    CH = 1600

    @functools.partial(
        pl.kernel,
        out_type=[
            jax.ShapeDtypeStruct((Ep,), F32),
            jax.ShapeDtypeStruct((NW, Mp), F32),
        ],
        mesh=_mesh(),
        compiler_params=pltpu.CompilerParams(needs_layout_passes=False),
        scratch_types=[
            pltpu.VMEM((Np,), F32),
            pltpu.VMEM((Mp,), F32),
            pltpu.VMEM((Mp,), F32),
            pltpu.VMEM((CH,), I32),
            pltpu.VMEM((CH,), I32),
            pltpu.VMEM((CH,), F32),
            pltpu.VMEM((16,), F32),
            pltpu.VMEM((16,), F32),
            pltpu.SemaphoreType.DMA,
        ],
    )
    def k(src_ref, he_ref, sx_ref, se_ref, mxs_ref, mxe_ref, ae_ref, as_ref,
          sx_v, se_v, asum_v, si_v, hi_v, ae_v, m1_v, m2_v, semi):
        wid = lax.axis_index("s") * NC + lax.axis_index("c")
        pltpu.sync_copy(sx_ref, sx_v)
        pltpu.sync_copy(se_ref, se_v)
        pltpu.sync_copy(mxs_ref, m1_v)
        pltpu.sync_copy(mxe_ref, m2_v)
        _zero_1d(asum_v, Mp)
        shift = jnp.maximum(m1_v[...] + m2_v[...], 0.0)

        base0 = wid * EPT

        def chunk(c, carry):
            b = base0 + c * CH
            d1 = pltpu.async_copy(src_ref.at[pl.ds(b, CH)], si_v, semi)
            d2 = pltpu.async_copy(he_ref.at[pl.ds(b, CH)], hi_v, semi)
            d1.wait()
            d2.wait()

            @functools.partial(plsc.parallel_loop, 0, CH // 16, unroll=2)
            def grp(g):
                o = g * 16
                ii = si_v[pl.ds(o, 16)]
                jj = hi_v[pl.ds(o, 16)]
                a = plsc.load_gather(sx_v, [ii]) + plsc.load_gather(se_v, [jj])
                a = jnp.where(a >= 0.0, a, 0.2 * a) - shift
                ae = jnp.exp(a)
                ae_v[pl.ds(o, 16)] = ae
                plsc.addupdate_scatter(asum_v, [jj], ae)

            pltpu.sync_copy(ae_v, ae_ref.at[pl.ds(b, CH)])
            return carry

        lax.fori_loop(0, EPT // CH, chunk, 0)
        pltpu.sync_copy(asum_v, as_ref.at[wid])

    return k(src_f, he_f, sx, se, mxs, mxe)


def _sc_pass(table, gidx2, sidx2, ae_f, rows_out, zrows):
    Ep = ae_f.shape[0]
    EPC = Ep // NC
    EPT = EPC // NS
    CH = 512
    RW = 64
    KB = CH // RW
    RPT = rows_out // NS

    @functools.partial(
        pl.kernel,
        out_type=jax.ShapeDtypeStruct((NC, rows_out, D), F32),
        mesh=_mesh(),
        compiler_params=pltpu.CompilerParams(
            needs_layout_passes=False, use_tc_tiling_on_sc=False),
        scratch_types=[
            pltpu.VMEM_SHARED((rows_out, D), F32),
            pltpu.VMEM((KB, RW), I32),
            pltpu.VMEM((KB, RW), I32),
            pltpu.VMEM((CH,), F32),
            pltpu.VMEM((CH, D), F32),
            pltpu.VMEM((CH, D), F32),
            pltpu.SemaphoreType.DMA,
            pltpu.SemaphoreType.DMA,
        ],
    )
    def k(t_ref, g_ref, s_ref, a_ref, z_ref, out_ref,
          acc_sh, gi_v, si_v, ae_v, rows_v, rows2_v, sem, semi):
        cid = lax.axis_index("c")
        sid = lax.axis_index("s")
        r0 = sid * RPT

        def zc(i, carry):
            pltpu.sync_copy(z_ref, acc_sh.at[pl.ds(r0 + i * 64, 64), :])
            return carry

        lax.fori_loop(0, RPT // 64, zc, 0)
        plsc.subcore_barrier()

        base0 = cid * EPC + sid * EPT
        lane = lax.iota(I32, 16)
        c0 = lane
        c1 = lane + 16
        c2 = lane + 32
        m8 = lane < 8

        def chunk(c, carry):
            b = base0 + c * CH
            brow = pl.multiple_of(b // RW, 8)
            d1 = pltpu.async_copy(g_ref.at[pl.ds(brow, KB), :], gi_v, semi)
            d2 = pltpu.async_copy(s_ref.at[pl.ds(brow, KB), :], si_v, semi)
            d3 = pltpu.async_copy(a_ref.at[pl.ds(b, CH)], ae_v, semi)
            d1.wait()
            d2.wait()
            d3.wait()
            descs = [
                pltpu.async_copy(
                    t_ref.at[gi_v.at[j]],
                    rows_v.at[pl.ds(j * RW, RW), :], sem)
                for j in range(KB)
            ]
            for d in descs:
                d.wait()

            @functools.partial(plsc.parallel_loop, 0, CH // 16, unroll=2)
            def grp16(g):
                eb = g * 16
                wv = ae_v[pl.ds(eb, 16)]
                for kk in range(16):
                    e = eb + kk
                    w = wv[kk]
                    ef = jnp.full((16,), 0, I32) + e
                    v0 = plsc.load_gather(rows_v, [ef, c0])
                    plsc.store_scatter(rows2_v, [ef, c0], v0 * w)
                    v1 = plsc.load_gather(rows_v, [ef, c1])
                    plsc.store_scatter(rows2_v, [ef, c1], v1 * w)
                    v2 = plsc.load_gather(rows_v, [ef, c2], mask=m8)
                    plsc.store_scatter(rows2_v, [ef, c2], v2 * w, mask=m8)

            for j in range(KB):
                pltpu.sync_copy(
                    rows2_v.at[pl.ds(j * RW, RW), :],
                    acc_sh.at[si_v.at[j]], add=True)
            return carry

        lax.fori_loop(0, EPT // CH, chunk, 0)
        plsc.subcore_barrier()

        def oc(i, carry):
            rr = r0 + i * 64
            pltpu.sync_copy(acc_sh.at[pl.ds(rr, 64), :],
                            out_ref.at[cid, pl.ds(rr, 64), :])
            return carry

        lax.fori_loop(0, RPT // 64, oc, 0)

    return k(table, gidx2, sidx2, ae_f, zrows)


def _sc_pass2(table2, gidx2, sidx2, ae_f, rows_out, zrows):
    Ep = ae_f.shape[0]
    EPT = Ep // NS
    CH = 512
    RW = 64
    KB = CH // RW
    DH = 24
    RPT = rows_out // NS

    @functools.partial(
        pl.kernel,
        out_type=jax.ShapeDtypeStruct((NC, rows_out, DH), F32),
        mesh=_mesh(),
        compiler_params=pltpu.CompilerParams(
            needs_layout_passes=False, use_tc_tiling_on_sc=False),
        scratch_types=[
            pltpu.VMEM_SHARED((rows_out, DH), F32),
            pltpu.VMEM((KB, RW), I32),
            pltpu.VMEM((KB, RW), I32),
            pltpu.VMEM((KB, RW), I32),
            pltpu.VMEM((CH,), F32),
            pltpu.VMEM((CH, DH), F32),
            pltpu.VMEM((CH, DH), F32),
            pltpu.SemaphoreType.DMA,
            pltpu.SemaphoreType.DMA,
        ],
    )
    def k(t_ref, g_ref, s_ref, a_ref, z_ref, out_ref,
          acc_sh, gi_v, ga_v, si_v, ae_v, rows_v, rows2_v, sem, semi):
        cid = lax.axis_index("c")
        sid = lax.axis_index("s")
        r0 = sid * RPT

        def zc(i, carry):
            pltpu.sync_copy(z_ref.at[:, pl.ds(0, DH)],
                            acc_sh.at[pl.ds(r0 + i * 64, 64), :])
            return carry

        lax.fori_loop(0, RPT // 64, zc, 0)
        plsc.subcore_barrier()

        base0 = sid * EPT
        lane = lax.iota(I32, 16)
        c0 = lane
        c1 = lane + 16
        m8 = lane < 8

        def chunk(c, carry):
            b = base0 + c * CH
            brow = pl.multiple_of(b // RW, 8)
            d1 = pltpu.async_copy(g_ref.at[pl.ds(brow, KB), :], gi_v, semi)
            d2 = pltpu.async_copy(s_ref.at[pl.ds(brow, KB), :], si_v, semi)
            d3 = pltpu.async_copy(a_ref.at[pl.ds(b, CH)], ae_v, semi)
            d1.wait()
            d2.wait()
            d3.wait()

            @functools.partial(plsc.parallel_loop, 0, CH // 16, unroll=2)
            def adjj(g):
                j = g // (RW // 16)
                o = (g % (RW // 16)) * 16
                ga_v[j, pl.ds(o, 16)] = gi_v[j, pl.ds(o, 16)] * 2 + cid

            descs = [
                pltpu.async_copy(
                    t_ref.at[ga_v.at[j]],
                    rows_v.at[pl.ds(j * RW, RW), :], sem)
                for j in range(KB)
            ]
            for d in descs:
                d.wait()

            @functools.partial(plsc.parallel_loop, 0, CH // 16, unroll=2)
            def grp16(g):
                eb = g * 16
                wv = ae_v[pl.ds(eb, 16)]
                for kk in range(16):
                    e = eb + kk
                    w = wv[kk]
                    ef = jnp.full((16,), 0, I32) + e
                    v0 = plsc.load_gather(rows_v, [ef, c0])
                    plsc.store_scatter(rows2_v, [ef, c0], v0 * w)
                    v1 = plsc.load_gather(rows_v, [ef, c1], mask=m8)
                    plsc.store_scatter(rows2_v, [ef, c1], v1 * w, mask=m8)

            for j in range(KB):
                pltpu.sync_copy(
                    rows2_v.at[pl.ds(j * RW, RW), :],
                    acc_sh.at[si_v.at[j]], add=True)
            return carry

        lax.fori_loop(0, EPT // CH, chunk, 0)
        plsc.subcore_barrier()

        def oc(i, carry):
            rr = r0 + i * 64
            pltpu.sync_copy(acc_sh.at[pl.ds(rr, 64), :],
                            out_ref.at[cid, pl.ds(rr, 64), :])
            return carry

        lax.fori_loop(0, RPT // 64, oc, 0)

    return k(table2, gidx2, sidx2, ae_f, zrows)


def _sc_pool(h_flat, batch_p, Np, Gp):
    H = 35
    RPT = Np // NW
    GW = Gp * D

    @functools.partial(
        pl.kernel,
        out_type=[
            jax.ShapeDtypeStruct((NW, GW), F32),
            jax.ShapeDtypeStruct((NW, Gp), F32),
        ],
        mesh=_mesh(),
        compiler_params=pltpu.CompilerParams(needs_layout_passes=False),
        scratch_types=[
            pltpu.VMEM((GW,), F32),
            pltpu.VMEM((Gp,), F32),
            pltpu.VMEM((RPT * H,), F32),
            pltpu.VMEM((RPT,), I32),
        ],
    )
    def k(h_ref, b_ref, op_ref, oc_ref, acc_v, cnt_v, hr_v, bt_v):
        wid = lax.axis_index("s") * NC + lax.axis_index("c")
        _zero_1d(acc_v, GW)
        _zero_1d(cnt_v, Gp)
        row0 = wid * RPT
        pltpu.sync_copy(h_ref.at[pl.ds(row0 * H, RPT * H)], hr_v)
        pltpu.sync_copy(b_ref.at[pl.ds(row0, RPT)], bt_v)

        lane = lax.iota(I32, 16)
        m3 = lane < 3
        ones = jnp.ones((16,), F32)

        def grp16(g, carry):
            rb0 = g * 16
            bv = bt_v[pl.ds(rb0, 16)]
            plsc.addupdate_scatter(cnt_v, [bv], ones)
            for kk in range(16):
                r = rb0 + kk
                gk = bv[kk]
                rb = (jnp.full((16,), 0, I32) + r * H) + lane
                ob = (jnp.full((16,), 0, I32) + gk * D) + lane
                v0 = plsc.load_gather(hr_v, [rb])
                plsc.addupdate_scatter(acc_v, [ob], v0)
                v1 = plsc.load_gather(hr_v, [rb + 16])
                plsc.addupdate_scatter(acc_v, [ob + 16], v1)
                v2 = plsc.load_gather(hr_v, [rb + 32], mask=m3)
                plsc.addupdate_scatter(acc_v, [ob + 32], v2, mask=m3)
            return carry

        lax.fori_loop(0, RPT // 16, grp16, 0)
        pltpu.sync_copy(acc_v, op_ref.at[wid])
        pltpu.sync_copy(cnt_v, oc_ref.at[wid])

    return k(h_flat, batch_p)


def _tc_embed(x_p, W, b, n_real):
    Np, IN = x_p.shape
    H = W.shape[1]

    def body(x_ref, w_ref, b_ref, o_ref):
        i = pl.program_id(0)
        h = jnp.dot(x_ref[...], w_ref[...], preferred_element_type=F32) + b_ref[...]
        rid = lax.broadcasted_iota(I32, (BN, H), 0) + i * BN
        o_ref[...] = jnp.where(rid < n_real, h, 0.0)

    return pl.pallas_call(
        body,
        grid=(Np // BN,),
        in_specs=[
            pl.BlockSpec((BN, IN), lambda i: (i, 0)),
            pl.BlockSpec(W.shape, lambda i: (0, 0)),
            pl.BlockSpec((1, H), lambda i: (0, 0)),
        ],
        out_specs=pl.BlockSpec((BN, H), lambda i: (i, 0)),
        out_shape=jax.ShapeDtypeStruct((Np, H), F32),
    )(x_p, W, b)


def _tc_xl(h, W, attx):
    Np, H = h.shape

    def body(h_ref, w_ref, a_ref, xl_ref, sx_ref, mx_ref):
        i = pl.program_id(0)
        xl = jnp.dot(h_ref[...], w_ref[...], preferred_element_type=F32)
        xl_ref[...] = jnp.concatenate(
            [xl, jnp.zeros((BN, D - H), F32)], axis=1)
        sx = jnp.dot(xl, a_ref[...], preferred_element_type=F32)
        sx_ref[...] = sx
        m = jnp.max(sx)

        @pl.when(i == 0)
        def _init():
            mx_ref[...] = jnp.full((1, 16), -3.4e38, F32)

        mx_ref[...] = jnp.maximum(mx_ref[...], m)

    return pl.pallas_call(
        body,
        grid=(Np // BN,),
        in_specs=[
            pl.BlockSpec((BN, H), lambda i: (i, 0)),
            pl.BlockSpec(W.shape, lambda i: (0, 0)),
            pl.BlockSpec((H, 1), lambda i: (0, 0)),
        ],
        out_specs=[
            pl.BlockSpec((BN, D), lambda i: (i, 0)),
            pl.BlockSpec((BN, 1), lambda i: (i, 0)),
            pl.BlockSpec((1, 16), lambda i: (0, 0)),
        ],
        out_shape=[
            jax.ShapeDtypeStruct((Np, D), F32),
            jax.ShapeDtypeStruct((Np, 1), F32),
            jax.ShapeDtypeStruct((1, 16), F32),
        ],
    )(h, W, attx)


def _tc_se(attr_p, W, atte):
    Mp, H = attr_p.shape

    def body(a_ref, w_ref, at_ref, se_ref, mx_ref):
        i = pl.program_id(0)
        el = jnp.dot(a_ref[...], w_ref[...], preferred_element_type=F32)
        se = jnp.dot(el, at_ref[...], preferred_element_type=F32)
        se_ref[...] = se
        m = jnp.max(se)

        @pl.when(i == 0)
        def _init():
            mx_ref[...] = jnp.full((1, 16), -3.4e38, F32)

        mx_ref[...] = jnp.maximum(mx_ref[...], m)

    return pl.pallas_call(
        body,
        grid=(Mp // BN,),
        in_specs=[
            pl.BlockSpec((BN, H), lambda i: (i, 0)),
            pl.BlockSpec(W.shape, lambda i: (0, 0)),
            pl.BlockSpec((H, 1), lambda i: (0, 0)),
        ],
        out_specs=[
            pl.BlockSpec((BN, 1), lambda i: (i, 0)),
            pl.BlockSpec((1, 16), lambda i: (0, 0)),
        ],
        out_shape=[
            jax.ShapeDtypeStruct((Mp, 1), F32),
            jax.ShapeDtypeStruct((1, 16), F32),
        ],
    )(attr_p, W, atte)


def _tc_T(acc1p, asum_p, cnthe_p):
    Mp = acc1p.shape[1]

    def body(a1_ref, as_ref, ch_ref, t_ref):
        acc = a1_ref[0] + a1_ref[1]
        asum = jnp.sum(as_ref[...], axis=0)
        cnt = jnp.sum(ch_ref[...], axis=0)
        be = jnp.where(cnt > 0, 1.0 / cnt, 0.0)
        rden = 1.0 / (asum + 1e-16)
        t = (be * (rden * rden))[:, None] * acc
        t48 = jnp.concatenate([t, jnp.zeros((BN, 8), F32)], axis=1)
        t_ref[...] = t48.reshape(BN, 2, 24)

    return pl.pallas_call(
        body,
        grid=(Mp // BN,),
        in_specs=[
            pl.BlockSpec((NC, BN, D), lambda i: (0, i, 0)),
            pl.BlockSpec((NW, BN), lambda i: (0, i)),
            pl.BlockSpec((NW, BN), lambda i: (0, i)),
        ],
        out_specs=pl.BlockSpec((BN, 2, 24), lambda i: (i, 0, 0)),
        out_shape=jax.ShapeDtypeStruct((Mp, 2, 24), F32),
    )(acc1p, asum_p, cnthe_p)


def _tc_F(acc2p, cntsrc_p, bias, n_real, H):
    Np = acc2p.shape[1]

    def body(a2_ref, cs_ref, b_ref, h_ref):
        i = pl.program_id(0)
        acc = jnp.concatenate([a2_ref[0], a2_ref[1]], axis=1)
        cnt = jnp.sum(cs_ref[...], axis=0)
        dv = jnp.where(cnt > 0, 1.0 / cnt, 0.0)
        h = dv[:, None] * acc[:, :H] + b_ref[...]
        rid = lax.broadcasted_iota(I32, (BN, H), 0) + i * BN
        h_ref[...] = jnp.where(rid < n_real, h, 0.0)

    return pl.pallas_call(
        body,
        grid=(Np // BN,),
        in_specs=[
            pl.BlockSpec((NC, BN, 24), lambda i: (0, i, 0)),
            pl.BlockSpec((NW, BN), lambda i: (0, i)),
            pl.BlockSpec((1, H), lambda i: (0, 0)),
        ],
        out_specs=pl.BlockSpec((BN, H), lambda i: (i, 0)),
        out_shape=jax.ShapeDtypeStruct((Np, H), F32),
    )(acc2p, cntsrc_p, bias)


def _tc_final(pp, cp, Wp_pad, bp, Wo, bo, G):
    def body(pp_ref, cp_ref, wp_ref, bp_ref, wo_ref, bo_ref, o_ref):
        pooled = jnp.sum(pp_ref[...], axis=0)
        cnt = jnp.sum(cp_ref[...], axis=0)
        pooled = pooled / jnp.maximum(cnt, 1.0)[:, None]
        z = jnp.dot(pooled, wp_ref[...], preferred_element_type=F32) + bp_ref[...]
        p = jnp.maximum(z, 0.0) + jnp.log(1.0 + jnp.exp(-jnp.abs(z)))
        o_ref[...] = jnp.dot(p, wo_ref[...], preferred_element_type=F32) + bo_ref[...]

    Gp = pp.shape[1]
    HOUT = Wp_pad.shape[1]
    return pl.pallas_call(
        body,
        in_specs=[
            pl.BlockSpec((NW, G, D), lambda: (0, 0, 0)),
            pl.BlockSpec((NW, G), lambda: (0, 0)),
            pl.BlockSpec(Wp_pad.shape, lambda: (0, 0)),
            pl.BlockSpec((1, HOUT), lambda: (0, 0)),
            pl.BlockSpec(Wo.shape, lambda: (0, 0)),
            pl.BlockSpec((1, 1), lambda: (0, 0)),
        ],
        out_specs=pl.BlockSpec((G, 1), lambda: (0, 0)),
        out_shape=jax.ShapeDtypeStruct((G, 1), F32),
    )(pp, cp, Wp_pad, bp, Wo, bo)


def kernel(x, hyperedge_index, hyperedge_attr, batch, W_embed, b_embed,
           W0, att0, bias0, W1, att1, bias1, W2, att2, bias2,
           W_proj, b_proj, W_out, b_out):
    N, IN = x.shape
    M, H = hyperedge_attr.shape
    E = hyperedge_index.shape[1]
    G = 512
    HOUT = W_proj.shape[1]

    Np = ((N + BN) // BN) * BN
    Mp = ((M + BN) // BN) * BN
    EPW_Q = NW * 1600
    Ep = ((E + EPW_Q - 1) // EPW_Q) * EPW_Q
    Gp = G + 32

    src = hyperedge_index[0]
    he = hyperedge_index[1]
    src_f = jnp.concatenate([src, jnp.full((Ep - E,), N, I32)])
    he_f = jnp.concatenate([he, jnp.full((Ep - E,), M, I32)])
    src2 = src_f.reshape(Ep // 64, 64)
    he2 = he_f.reshape(Ep // 64, 64)
    x_p = jnp.pad(x, ((0, Np - N), (0, 0)))
    attr_p = jnp.pad(hyperedge_attr, ((0, Mp - M), (0, 0)))
    batch_p = jnp.concatenate([batch, jnp.full((Np - N,), G, I32)])
    zrows = jnp.zeros((64, D), F32)

    cn_p, cm_p = _sc_degree(src_f, he_f, Np, Mp)

    h = _tc_embed(x_p, W_embed, b_embed.reshape(1, H), N)

    layers = ((W0, att0, bias0), (W1, att1, bias1), (W2, att2, bias2))
    for (W, att, bias) in layers:
        attx = att[:H].reshape(H, 1)
        atte = att[H:].reshape(H, 1)
        xlp, sx, mxs = _tc_xl(h, W, attx)
        se, mxe = _tc_se(attr_p, W, atte)
        ae_f, as_p = _sc_alpha(src_f, he_f, sx.reshape(Np), se.reshape(Mp),
                               mxs.reshape(16), mxe.reshape(16))
        acc1p = _sc_pass(xlp, src2, he2, ae_f, Mp, zrows)
        T2 = _tc_T(acc1p, as_p, cm_p).reshape(2 * Mp, 24)
        acc2p = _sc_pass2(T2, he2, src2, ae_f, Np, zrows)
        h = _tc_F(acc2p, cn_p, bias.reshape(1, H), N, H)

    pp_f, cp = _sc_pool(h.reshape(Np * H), batch_p, Np, Gp)
    pp = pp_f.reshape(NW, Gp, D)[:, :G, :]
    Wp_pad = jnp.pad(W_proj, ((0, D - H), (0, 0)))
    return _tc_final(pp, cp[:, :G], Wp_pad, b_proj.reshape(1, HOUT),
                     W_out, b_out.reshape(1, 1), G)


# speedup vs baseline: 5.1580x; 5.1580x over previous
import functools

import jax
import jax.numpy as jnp
from jax import lax
from jax.experimental import pallas as pl
from jax.experimental.pallas import tpu as pltpu
from jax.experimental.pallas import tpu_sc as plsc

F32 = jnp.float32
I32 = jnp.int32

NC, NS, LANES = 2, 16, 16
NW = NC * NS
BN = 512
D = 40


def _mesh():
    return plsc.VectorSubcoreMesh(core_axis_name="c", subcore_axis_name="s")


def _zero_1d(ref, n):
    z16 = jnp.zeros((16,), F32)

    def body(i, c):
        ref[pl.ds(i * 16, 16)] = z16
        return c

    lax.fori_loop(0, n // 16, body, 0)


def _sc_degree(src_f, he_f, Np, Mp):
    Ep = src_f.shape[0]
    EPT = Ep // NW
    CH = 1600

    @functools.partial(
        pl.kernel,
        out_type=[
            jax.ShapeDtypeStruct((NW, Np), F32),
            jax.ShapeDtypeStruct((NW, Mp), F32),
        ],
        mesh=_mesh(),
        compiler_params=pltpu.CompilerParams(needs_layout_passes=False),
        scratch_types=[
            pltpu.VMEM((Np,), F32),
            pltpu.VMEM((Mp,), F32),
            pltpu.VMEM((CH,), I32),
            pltpu.VMEM((CH,), I32),
            pltpu.SemaphoreType.DMA,
        ],
    )
    def k(src_ref, he_ref, on_ref, om_ref, cn_v, cm_v, si_v, hi_v, semi):
        wid = lax.axis_index("s") * NC + lax.axis_index("c")
        _zero_1d(cn_v, Np)
        _zero_1d(cm_v, Mp)
        ones = jnp.ones((16,), F32)
        base0 = wid * EPT

        def chunk(c, carry):
            b = base0 + c * CH
            d1 = pltpu.async_copy(src_ref.at[pl.ds(b, CH)], si_v, semi)
            d2 = pltpu.async_copy(he_ref.at[pl.ds(b, CH)], hi_v, semi)
            d1.wait()
            d2.wait()

            def grp(g, cg):
                o = g * 16
                plsc.addupdate_scatter(cn_v, [si_v[pl.ds(o, 16)]], ones)
                plsc.addupdate_scatter(cm_v, [hi_v[pl.ds(o, 16)]], ones)
                return cg

            lax.fori_loop(0, CH // 16, grp, 0)
            return carry

        lax.fori_loop(0, EPT // CH, chunk, 0)
        pltpu.sync_copy(cn_v, on_ref.at[wid])
        pltpu.sync_copy(cm_v, om_ref.at[wid])

    return k(src_f, he_f)


def _sc_alpha(src_f, he_f, sx, se, mxs, mxe):
    Ep = src_f.shape[0]
    Np = sx.shape[0]
    Mp = se.shape[0]
    EPT = Ep // NW
    CH = 1600

    @functools.partial(
        pl.kernel,
        out_type=[
            jax.ShapeDtypeStruct((Ep,), F32),
            jax.ShapeDtypeStruct((NW, Mp), F32),
        ],
        mesh=_mesh(),
        compiler_params=pltpu.CompilerParams(needs_layout_passes=False),
        scratch_types=[
            pltpu.VMEM((Np,), F32),
            pltpu.VMEM((Mp,), F32),
            pltpu.VMEM((Mp,), F32),
            pltpu.VMEM((CH,), I32),
            pltpu.VMEM((CH,), I32),
            pltpu.VMEM((CH,), F32),
            pltpu.VMEM((16,), F32),
            pltpu.VMEM((16,), F32),
            pltpu.SemaphoreType.DMA,
        ],
    )
    def k(src_ref, he_ref, sx_ref, se_ref, mxs_ref, mxe_ref, ae_ref, as_ref,
          sx_v, se_v, asum_v, si_v, hi_v, ae_v, m1_v, m2_v, semi):
        wid = lax.axis_index("s") * NC + lax.axis_index("c")
        pltpu.sync_copy(sx_ref, sx_v)
        pltpu.sync_copy(se_ref, se_v)
        pltpu.sync_copy(mxs_ref, m1_v)
        pltpu.sync_copy(mxe_ref, m2_v)
        _zero_1d(asum_v, Mp)
        shift = jnp.maximum(m1_v[...] + m2_v[...], 0.0)

        base0 = wid * EPT

        def chunk(c, carry):
            b = base0 + c * CH
            d1 = pltpu.async_copy(src_ref.at[pl.ds(b, CH)], si_v, semi)
            d2 = pltpu.async_copy(he_ref.at[pl.ds(b, CH)], hi_v, semi)
            d1.wait()
            d2.wait()

            def grp(g, cg):
                o = g * 16
                ii = si_v[pl.ds(o, 16)]
                jj = hi_v[pl.ds(o, 16)]
                a = plsc.load_gather(sx_v, [ii]) + plsc.load_gather(se_v, [jj])
                a = jnp.where(a >= 0.0, a, 0.2 * a) - shift
                ae = jnp.exp(a)
                ae_v[pl.ds(o, 16)] = ae
                plsc.addupdate_scatter(asum_v, [jj], ae)
                return cg

            lax.fori_loop(0, CH // 16, grp, 0)
            pltpu.sync_copy(ae_v, ae_ref.at[pl.ds(b, CH)])
            return carry

        lax.fori_loop(0, EPT // CH, chunk, 0)
        pltpu.sync_copy(asum_v, as_ref.at[wid])

    return k(src_f, he_f, sx, se, mxs, mxe)


def _sc_pass(table, gidx2, sidx2, ae_f, rows_out, zrows):
    Ep = ae_f.shape[0]
    EPC = Ep // NC
    EPT = EPC // NS
    CH = 512
    RW = 64
    KB = CH // RW
    RPT = rows_out // NS

    @functools.partial(
        pl.kernel,
        out_type=jax.ShapeDtypeStruct((NC, rows_out, D), F32),
        mesh=_mesh(),
        compiler_params=pltpu.CompilerParams(
            needs_layout_passes=False, use_tc_tiling_on_sc=False),
        scratch_types=[
            pltpu.VMEM_SHARED((rows_out, D), F32),
            pltpu.VMEM((KB, RW), I32),
            pltpu.VMEM((KB, RW), I32),
            pltpu.VMEM((CH,), F32),
            pltpu.VMEM((CH, D), F32),
            pltpu.VMEM((CH, D), F32),
            pltpu.SemaphoreType.DMA,
            pltpu.SemaphoreType.DMA,
        ],
    )
    def k(t_ref, g_ref, s_ref, a_ref, z_ref, out_ref,
          acc_sh, gi_v, si_v, ae_v, rows_v, rows2_v, sem, semi):
        cid = lax.axis_index("c")
        sid = lax.axis_index("s")
        r0 = sid * RPT

        def zc(i, carry):
            pltpu.sync_copy(z_ref, acc_sh.at[pl.ds(r0 + i * 64, 64), :])
            return carry

        lax.fori_loop(0, RPT // 64, zc, 0)
        plsc.subcore_barrier()

        base0 = cid * EPC + sid * EPT
        lane = lax.iota(I32, 16)
        c0 = lane
        c1 = lane + 16
        c2 = lane + 32
        m8 = lane < 8

        def chunk(c, carry):
            b = base0 + c * CH
            brow = pl.multiple_of(b // RW, 8)
            d1 = pltpu.async_copy(g_ref.at[pl.ds(brow, KB), :], gi_v, semi)
            d2 = pltpu.async_copy(s_ref.at[pl.ds(brow, KB), :], si_v, semi)
            d3 = pltpu.async_copy(a_ref.at[pl.ds(b, CH)], ae_v, semi)
            d1.wait()
            d2.wait()
            d3.wait()
            descs = [
                pltpu.async_copy(
                    t_ref.at[gi_v.at[j]],
                    rows_v.at[pl.ds(j * RW, RW), :], sem)
                for j in range(KB)
            ]
            for d in descs:
                d.wait()

            def grp16(g, cg):
                eb = g * 16
                wv = ae_v[pl.ds(eb, 16)]
                for kk in range(16):
                    e = eb + kk
                    w = wv[kk]
                    ef = jnp.full((16,), 0, I32) + e
                    v0 = plsc.load_gather(rows_v, [ef, c0])
                    plsc.store_scatter(rows2_v, [ef, c0], v0 * w)
                    v1 = plsc.load_gather(rows_v, [ef, c1])
                    plsc.store_scatter(rows2_v, [ef, c1], v1 * w)
                    v2 = plsc.load_gather(rows_v, [ef, c2], mask=m8)
                    plsc.store_scatter(rows2_v, [ef, c2], v2 * w, mask=m8)
                return cg

            lax.fori_loop(0, CH // 16, grp16, 0)
            for j in range(KB):
                pltpu.sync_copy(
                    rows2_v.at[pl.ds(j * RW, RW), :],
                    acc_sh.at[si_v.at[j]], add=True)
            return carry

        lax.fori_loop(0, EPT // CH, chunk, 0)
        plsc.subcore_barrier()

        def oc(i, carry):
            rr = r0 + i * 64
            pltpu.sync_copy(acc_sh.at[pl.ds(rr, 64), :],
                            out_ref.at[cid, pl.ds(rr, 64), :])
            return carry

        lax.fori_loop(0, RPT // 64, oc, 0)

    return k(table, gidx2, sidx2, ae_f, zrows)


def _sc_pass2(table2, gidx2, sidx2, ae_f, rows_out, zrows):
    Ep = ae_f.shape[0]
    EPT = Ep // NS
    CH = 512
    RW = 64
    KB = CH // RW
    DH = 24
    RPT = rows_out // NS

    @functools.partial(
        pl.kernel,
        out_type=jax.ShapeDtypeStruct((NC, rows_out, DH), F32),
        mesh=_mesh(),
        compiler_params=pltpu.CompilerParams(
            needs_layout_passes=False, use_tc_tiling_on_sc=False),
        scratch_types=[
            pltpu.VMEM_SHARED((rows_out, DH), F32),
            pltpu.VMEM((KB, RW), I32),
            pltpu.VMEM((KB, RW), I32),
            pltpu.VMEM((KB, RW), I32),
            pltpu.VMEM((CH,), F32),
            pltpu.VMEM((CH, DH), F32),
            pltpu.VMEM((CH, DH), F32),
            pltpu.SemaphoreType.DMA,
            pltpu.SemaphoreType.DMA,
        ],
    )
    def k(t_ref, g_ref, s_ref, a_ref, z_ref, out_ref,
          acc_sh, gi_v, ga_v, si_v, ae_v, rows_v, rows2_v, sem, semi):
        cid = lax.axis_index("c")
        sid = lax.axis_index("s")
        r0 = sid * RPT

        def zc(i, carry):
            pltpu.sync_copy(z_ref.at[:, pl.ds(0, DH)],
                            acc_sh.at[pl.ds(r0 + i * 64, 64), :])
            return carry

        lax.fori_loop(0, RPT // 64, zc, 0)
        plsc.subcore_barrier()

        base0 = sid * EPT
        lane = lax.iota(I32, 16)
        c0 = lane
        c1 = lane + 16
        m8 = lane < 8

        def chunk(c, carry):
            b = base0 + c * CH
            brow = pl.multiple_of(b // RW, 8)
            d1 = pltpu.async_copy(g_ref.at[pl.ds(brow, KB), :], gi_v, semi)
            d2 = pltpu.async_copy(s_ref.at[pl.ds(brow, KB), :], si_v, semi)
            d3 = pltpu.async_copy(a_ref.at[pl.ds(b, CH)], ae_v, semi)
            d1.wait()
            d2.wait()
            d3.wait()

            def adjj(g, cg):
                j = g // (RW // 16)
                o = (g % (RW // 16)) * 16
                ga_v[j, pl.ds(o, 16)] = gi_v[j, pl.ds(o, 16)] * 2 + cid
                return cg

            lax.fori_loop(0, CH // 16, adjj, 0)
            descs = [
                pltpu.async_copy(
                    t_ref.at[ga_v.at[j]],
                    rows_v.at[pl.ds(j * RW, RW), :], sem)
                for j in range(KB)
            ]
            for d in descs:
                d.wait()

            def grp16(g, cg):
                eb = g * 16
                wv = ae_v[pl.ds(eb, 16)]
                for kk in range(16):
                    e = eb + kk
                    w = wv[kk]
                    ef = jnp.full((16,), 0, I32) + e
                    v0 = plsc.load_gather(rows_v, [ef, c0])
                    plsc.store_scatter(rows2_v, [ef, c0], v0 * w)
                    v1 = plsc.load_gather(rows_v, [ef, c1], mask=m8)
                    plsc.store_scatter(rows2_v, [ef, c1], v1 * w, mask=m8)
                return cg

            lax.fori_loop(0, CH // 16, grp16, 0)
            for j in range(KB):
                pltpu.sync_copy(
                    rows2_v.at[pl.ds(j * RW, RW), :],
                    acc_sh.at[si_v.at[j]], add=True)
            return carry

        lax.fori_loop(0, EPT // CH, chunk, 0)
        plsc.subcore_barrier()

        def oc(i, carry):
            rr = r0 + i * 64
            pltpu.sync_copy(acc_sh.at[pl.ds(rr, 64), :],
                            out_ref.at[cid, pl.ds(rr, 64), :])
            return carry

        lax.fori_loop(0, RPT // 64, oc, 0)

    return k(table2, gidx2, sidx2, ae_f, zrows)


def _sc_pool(h_flat, batch_p, Np, Gp):
    H = 35
    RPT = Np // NW
    GW = Gp * D

    @functools.partial(
        pl.kernel,
        out_type=[
            jax.ShapeDtypeStruct((NW, GW), F32),
            jax.ShapeDtypeStruct((NW, Gp), F32),
        ],
        mesh=_mesh(),
        compiler_params=pltpu.CompilerParams(needs_layout_passes=False),
        scratch_types=[
            pltpu.VMEM((GW,), F32),
            pltpu.VMEM((Gp,), F32),
            pltpu.VMEM((RPT * H,), F32),
            pltpu.VMEM((RPT,), I32),
        ],
    )
    def k(h_ref, b_ref, op_ref, oc_ref, acc_v, cnt_v, hr_v, bt_v):
        wid = lax.axis_index("s") * NC + lax.axis_index("c")
        _zero_1d(acc_v, GW)
        _zero_1d(cnt_v, Gp)
        row0 = wid * RPT
        pltpu.sync_copy(h_ref.at[pl.ds(row0 * H, RPT * H)], hr_v)
        pltpu.sync_copy(b_ref.at[pl.ds(row0, RPT)], bt_v)

        lane = lax.iota(I32, 16)
        m3 = lane < 3
        ones = jnp.ones((16,), F32)

        def grp16(g, carry):
            rb0 = g * 16
            bv = bt_v[pl.ds(rb0, 16)]
            plsc.addupdate_scatter(cnt_v, [bv], ones)
            for kk in range(16):
                r = rb0 + kk
                gk = bv[kk]
                rb = (jnp.full((16,), 0, I32) + r * H) + lane
                ob = (jnp.full((16,), 0, I32) + gk * D) + lane
                v0 = plsc.load_gather(hr_v, [rb])
                plsc.addupdate_scatter(acc_v, [ob], v0)
                v1 = plsc.load_gather(hr_v, [rb + 16])
                plsc.addupdate_scatter(acc_v, [ob + 16], v1)
                v2 = plsc.load_gather(hr_v, [rb + 32], mask=m3)
                plsc.addupdate_scatter(acc_v, [ob + 32], v2, mask=m3)
            return carry

        lax.fori_loop(0, RPT // 16, grp16, 0)
        pltpu.sync_copy(acc_v, op_ref.at[wid])
        pltpu.sync_copy(cnt_v, oc_ref.at[wid])

    return k(h_flat, batch_p)


def _tc_embed(x_p, W, b, n_real):
    Np, IN = x_p.shape
    H = W.shape[1]

    def body(x_ref, w_ref, b_ref, o_ref):
        i = pl.program_id(0)
        h = jnp.dot(x_ref[...], w_ref[...], preferred_element_type=F32) + b_ref[...]
        rid = lax.broadcasted_iota(I32, (BN, H), 0) + i * BN
        o_ref[...] = jnp.where(rid < n_real, h, 0.0)

    return pl.pallas_call(
        body,
        grid=(Np // BN,),
        in_specs=[
            pl.BlockSpec((BN, IN), lambda i: (i, 0)),
            pl.BlockSpec(W.shape, lambda i: (0, 0)),
            pl.BlockSpec((1, H), lambda i: (0, 0)),
        ],
        out_specs=pl.BlockSpec((BN, H), lambda i: (i, 0)),
        out_shape=jax.ShapeDtypeStruct((Np, H), F32),
    )(x_p, W, b)


def _tc_xl(h, W, attx):
    Np, H = h.shape

    def body(h_ref, w_ref, a_ref, xl_ref, sx_ref, mx_ref):
        i = pl.program_id(0)
        xl = jnp.dot(h_ref[...], w_ref[...], preferred_element_type=F32)
        xl_ref[...] = jnp.concatenate(
            [xl, jnp.zeros((BN, D - H), F32)], axis=1)
        sx = jnp.dot(xl, a_ref[...], preferred_element_type=F32)
        sx_ref[...] = sx
        m = jnp.max(sx)

        @pl.when(i == 0)
        def _init():
            mx_ref[...] = jnp.full((1, 16), -3.4e38, F32)

        mx_ref[...] = jnp.maximum(mx_ref[...], m)

    return pl.pallas_call(
        body,
        grid=(Np // BN,),
        in_specs=[
            pl.BlockSpec((BN, H), lambda i: (i, 0)),
            pl.BlockSpec(W.shape, lambda i: (0, 0)),
            pl.BlockSpec((H, 1), lambda i: (0, 0)),
        ],
        out_specs=[
            pl.BlockSpec((BN, D), lambda i: (i, 0)),
            pl.BlockSpec((BN, 1), lambda i: (i, 0)),
            pl.BlockSpec((1, 16), lambda i: (0, 0)),
        ],
        out_shape=[
            jax.ShapeDtypeStruct((Np, D), F32),
            jax.ShapeDtypeStruct((Np, 1), F32),
            jax.ShapeDtypeStruct((1, 16), F32),
        ],
    )(h, W, attx)


def _tc_se(attr_p, W, atte):
    Mp, H = attr_p.shape

    def body(a_ref, w_ref, at_ref, se_ref, mx_ref):
        i = pl.program_id(0)
        el = jnp.dot(a_ref[...], w_ref[...], preferred_element_type=F32)
        se = jnp.dot(el, at_ref[...], preferred_element_type=F32)
        se_ref[...] = se
        m = jnp.max(se)

        @pl.when(i == 0)
        def _init():
            mx_ref[...] = jnp.full((1, 16), -3.4e38, F32)

        mx_ref[...] = jnp.maximum(mx_ref[...], m)

    return pl.pallas_call(
        body,
        grid=(Mp // BN,),
        in_specs=[
            pl.BlockSpec((BN, H), lambda i: (i, 0)),
            pl.BlockSpec(W.shape, lambda i: (0, 0)),
            pl.BlockSpec((H, 1), lambda i: (0, 0)),
        ],
        out_specs=[
            pl.BlockSpec((BN, 1), lambda i: (i, 0)),
            pl.BlockSpec((1, 16), lambda i: (0, 0)),
        ],
        out_shape=[
            jax.ShapeDtypeStruct((Mp, 1), F32),
            jax.ShapeDtypeStruct((1, 16), F32),
        ],
    )(attr_p, W, atte)


def _tc_T(acc1p, asum_p, cnthe_p):
    Mp = acc1p.shape[1]

    def body(a1_ref, as_ref, ch_ref, t_ref):
        acc = a1_ref[0] + a1_ref[1]
        asum = jnp.sum(as_ref[...], axis=0)
        cnt = jnp.sum(ch_ref[...], axis=0)
        be = jnp.where(cnt > 0, 1.0 / cnt, 0.0)
        rden = 1.0 / (asum + 1e-16)
        t = (be * (rden * rden))[:, None] * acc
        t48 = jnp.concatenate([t, jnp.zeros((BN, 8), F32)], axis=1)
        t_ref[...] = t48.reshape(BN, 2, 24)

    return pl.pallas_call(
        body,
        grid=(Mp // BN,),
        in_specs=[
            pl.BlockSpec((NC, BN, D), lambda i: (0, i, 0)),
            pl.BlockSpec((NW, BN), lambda i: (0, i)),
            pl.BlockSpec((NW, BN), lambda i: (0, i)),
        ],
        out_specs=pl.BlockSpec((BN, 2, 24), lambda i: (i, 0, 0)),
        out_shape=jax.ShapeDtypeStruct((Mp, 2, 24), F32),
    )(acc1p, asum_p, cnthe_p)


def _tc_F(acc2p, cntsrc_p, bias, n_real, H):
    Np = acc2p.shape[1]

    def body(a2_ref, cs_ref, b_ref, h_ref):
        i = pl.program_id(0)
        acc = jnp.concatenate([a2_ref[0], a2_ref[1]], axis=1)
        cnt = jnp.sum(cs_ref[...], axis=0)
        dv = jnp.where(cnt > 0, 1.0 / cnt, 0.0)
        h = dv[:, None] * acc[:, :H] + b_ref[...]
        rid = lax.broadcasted_iota(I32, (BN, H), 0) + i * BN
        h_ref[...] = jnp.where(rid < n_real, h, 0.0)

    return pl.pallas_call(
        body,
        grid=(Np // BN,),
        in_specs=[
            pl.BlockSpec((NC, BN, 24), lambda i: (0, i, 0)),
            pl.BlockSpec((NW, BN), lambda i: (0, i)),
            pl.BlockSpec((1, H), lambda i: (0, 0)),
        ],
        out_specs=pl.BlockSpec((BN, H), lambda i: (i, 0)),
        out_shape=jax.ShapeDtypeStruct((Np, H), F32),
    )(acc2p, cntsrc_p, bias)


def _tc_final(pp, cp, Wp_pad, bp, Wo, bo, G):
    def body(pp_ref, cp_ref, wp_ref, bp_ref, wo_ref, bo_ref, o_ref):
        pooled = jnp.sum(pp_ref[...], axis=0)
        cnt = jnp.sum(cp_ref[...], axis=0)
        pooled = pooled / jnp.maximum(cnt, 1.0)[:, None]
        z = jnp.dot(pooled, wp_ref[...], preferred_element_type=F32) + bp_ref[...]
        p = jnp.maximum(z, 0.0) + jnp.log(1.0 + jnp.exp(-jnp.abs(z)))
        o_ref[...] = jnp.dot(p, wo_ref[...], preferred_element_type=F32) + bo_ref[...]

    Gp = pp.shape[1]
    HOUT = Wp_pad.shape[1]
    return pl.pallas_call(
        body,
        in_specs=[
            pl.BlockSpec((NW, G, D), lambda: (0, 0, 0)),
            pl.BlockSpec((NW, G), lambda: (0, 0)),
            pl.BlockSpec(Wp_pad.shape, lambda: (0, 0)),
            pl.BlockSpec((1, HOUT), lambda: (0, 0)),
            pl.BlockSpec(Wo.shape, lambda: (0, 0)),
            pl.BlockSpec((1, 1), lambda: (0, 0)),
        ],
        out_specs=pl.BlockSpec((G, 1), lambda: (0, 0)),
        out_shape=jax.ShapeDtypeStruct((G, 1), F32),
    )(pp, cp, Wp_pad, bp, Wo, bo)


def kernel(x, hyperedge_index, hyperedge_attr, batch, W_embed, b_embed,
           W0, att0, bias0, W1, att1, bias1, W2, att2, bias2,
           W_proj, b_proj, W_out, b_out):
    N, IN = x.shape
    M, H = hyperedge_attr.shape
    E = hyperedge_index.shape[1]
    G = 512
    HOUT = W_proj.shape[1]

    Np = ((N + BN) // BN) * BN
    Mp = ((M + BN) // BN) * BN
    EPW_Q = NW * 1600
    Ep = ((E + EPW_Q - 1) // EPW_Q) * EPW_Q
    Gp = G + 32

    src = hyperedge_index[0]
    he = hyperedge_index[1]
    src_f = jnp.concatenate([src, jnp.full((Ep - E,), N, I32)])
    he_f = jnp.concatenate([he, jnp.full((Ep - E,), M, I32)])
    src2 = src_f.reshape(Ep // 64, 64)
    he2 = he_f.reshape(Ep // 64, 64)
    x_p = jnp.pad(x, ((0, Np - N), (0, 0)))
    attr_p = jnp.pad(hyperedge_attr, ((0, Mp - M), (0, 0)))
    batch_p = jnp.concatenate([batch, jnp.full((Np - N,), G, I32)])
    zrows = jnp.zeros((64, D), F32)

    cn_p, cm_p = _sc_degree(src_f, he_f, Np, Mp)

    h = _tc_embed(x_p, W_embed, b_embed.reshape(1, H), N)

    layers = ((W0, att0, bias0), (W1, att1, bias1), (W2, att2, bias2))
    for (W, att, bias) in layers:
        attx = att[:H].reshape(H, 1)
        atte = att[H:].reshape(H, 1)
        xlp, sx, mxs = _tc_xl(h, W, attx)
        se, mxe = _tc_se(attr_p, W, atte)
        ae_f, as_p = _sc_alpha(src_f, he_f, sx.reshape(Np), se.reshape(Mp),
                               mxs.reshape(16), mxe.reshape(16))
        acc1p = _sc_pass(xlp, src2, he2, ae_f, Mp, zrows)
        T2 = _tc_T(acc1p, as_p, cm_p).reshape(2 * Mp, 24)
        acc2p = _sc_pass2(T2, he2, src2, ae_f, Np, zrows)
        h = _tc_F(acc2p, cn_p, bias.reshape(1, H), N, H)

    pp_f, cp = _sc_pool(h.reshape(Np * H), batch_p, Np, Gp)
    pp = pp_f.reshape(NW, Gp, D)[:, :G, :]
    Wp_pad = jnp.pad(W_proj, ((0, D - H), (0, 0)))
    return _tc_final(pp, cp[:, :G], Wp_pad, b_proj.reshape(1, HOUT),
                     W_out, b_out.reshape(1, 1), G)


# speedup vs baseline: 5.7828x; 1.1211x over previous
import functools

import jax
import jax.numpy as jnp
from jax import lax
from jax.experimental import pallas as pl
from jax.experimental.pallas import tpu as pltpu
from jax.experimental.pallas import tpu_sc as plsc

F32 = jnp.float32
I32 = jnp.int32

NC, NS, LANES = 2, 16, 16
NW = NC * NS
BN = 512
D = 40


def _mesh():
    return plsc.VectorSubcoreMesh(core_axis_name="c", subcore_axis_name="s")


def _zero_1d(ref, n):
    z16 = jnp.zeros((16,), F32)

    def body(i, c):
        ref[pl.ds(i * 16, 16)] = z16
        return c

    lax.fori_loop(0, n // 16, body, 0)


def _sc_degree(src_f, he_f, Np, Mp):
    Ep = src_f.shape[0]
    EPT = Ep // NW
    CH = 1600

    @functools.partial(
        pl.kernel,
        out_type=[
            jax.ShapeDtypeStruct((NW, Np), F32),
            jax.ShapeDtypeStruct((NW, Mp), F32),
        ],
        mesh=_mesh(),
        compiler_params=pltpu.CompilerParams(needs_layout_passes=False),
        scratch_types=[
            pltpu.VMEM((Np,), F32),
            pltpu.VMEM((Mp,), F32),
            pltpu.VMEM((CH,), I32),
            pltpu.VMEM((CH,), I32),
            pltpu.SemaphoreType.DMA,
        ],
    )
    def k(src_ref, he_ref, on_ref, om_ref, cn_v, cm_v, si_v, hi_v, semi):
        wid = lax.axis_index("s") * NC + lax.axis_index("c")
        _zero_1d(cn_v, Np)
        _zero_1d(cm_v, Mp)
        ones = jnp.ones((16,), F32)
        base0 = wid * EPT

        def chunk(c, carry):
            b = base0 + c * CH
            d1 = pltpu.async_copy(src_ref.at[pl.ds(b, CH)], si_v, semi)
            d2 = pltpu.async_copy(he_ref.at[pl.ds(b, CH)], hi_v, semi)
            d1.wait()
            d2.wait()

            def grp(g, cg):
                o = g * 16
                plsc.addupdate_scatter(cn_v, [si_v[pl.ds(o, 16)]], ones)
                plsc.addupdate_scatter(cm_v, [hi_v[pl.ds(o, 16)]], ones)
                return cg

            lax.fori_loop(0, CH // 16, grp, 0)
            return carry

        lax.fori_loop(0, EPT // CH, chunk, 0)
        pltpu.sync_copy(cn_v, on_ref.at[wid])
        pltpu.sync_copy(cm_v, om_ref.at[wid])

    return k(src_f, he_f)


def _sc_alpha(src_f, he_f, sx, se, mxs, mxe):
    Ep = src_f.shape[0]
    Np = sx.shape[0]
    Mp = se.shape[0]
    EPT = Ep // NW
    CH = 1600

    @functools.partial(
        pl.kernel,
        out_type=[
            jax.ShapeDtypeStruct((Ep,), F32),
            jax.ShapeDtypeStruct((NW, Mp), F32),
        ],
        mesh=_mesh(),
        compiler_params=pltpu.CompilerParams(needs_layout_passes=False),
        scratch_types=[
            pltpu.VMEM((Np,), F32),
            pltpu.VMEM((Mp,), F32),
            pltpu.VMEM((Mp,), F32),
            pltpu.VMEM((CH,), I32),
            pltpu.VMEM((CH,), I32),
            pltpu.VMEM((CH,), F32),
            pltpu.VMEM((16,), F32),
            pltpu.VMEM((16,), F32),
            pltpu.SemaphoreType.DMA,
        ],
    )
    def k(src_ref, he_ref, sx_ref, se_ref, mxs_ref, mxe_ref, ae_ref, as_ref,
          sx_v, se_v, asum_v, si_v, hi_v, ae_v, m1_v, m2_v, semi):
        wid = lax.axis_index("s") * NC + lax.axis_index("c")
        pltpu.sync_copy(sx_ref, sx_v)
        pltpu.sync_copy(se_ref, se_v)
        pltpu.sync_copy(mxs_ref, m1_v)
        pltpu.sync_copy(mxe_ref, m2_v)
        _zero_1d(asum_v, Mp)
        shift = jnp.maximum(m1_v[...] + m2_v[...], 0.0)

        base0 = wid * EPT

        def chunk(c, carry):
            b = base0 + c * CH
            d1 = pltpu.async_copy(src_ref.at[pl.ds(b, CH)], si_v, semi)
            d2 = pltpu.async_copy(he_ref.at[pl.ds(b, CH)], hi_v, semi)
            d1.wait()
            d2.wait()

            def grp(g, cg):
                o = g * 16
                ii = si_v[pl.ds(o, 16)]
                jj = hi_v[pl.ds(o, 16)]
                a = plsc.load_gather(sx_v, [ii]) + plsc.load_gather(se_v, [jj])
                a = jnp.where(a >= 0.0, a, 0.2 * a) - shift
                ae = jnp.exp(a)
                ae_v[pl.ds(o, 16)] = ae
                plsc.addupdate_scatter(asum_v, [jj], ae)
                return cg

            lax.fori_loop(0, CH // 16, grp, 0)
            pltpu.sync_copy(ae_v, ae_ref.at[pl.ds(b, CH)])
            return carry

        lax.fori_loop(0, EPT // CH, chunk, 0)
        pltpu.sync_copy(asum_v, as_ref.at[wid])

    return k(src_f, he_f, sx, se, mxs, mxe)


def _sc_passS(ta, tb, gidx2, sidx2, ae_f, rows_acc, zrows):
    Ep = ae_f.shape[0]
    rows_tbl = ta.shape[0]
    EPT = Ep // NS
    CH = 512
    RW = 64
    KB = CH // RW
    DH = 24
    RT = rows_tbl // NS
    RA = rows_acc // NS

    @functools.partial(
        pl.kernel,
        out_type=jax.ShapeDtypeStruct((NC, rows_acc, DH), F32),
        mesh=_mesh(),
        compiler_params=pltpu.CompilerParams(
            needs_layout_passes=False, use_tc_tiling_on_sc=False),
        scratch_types=[
            pltpu.VMEM_SHARED((rows_tbl, DH), F32),
            pltpu.VMEM_SHARED((rows_acc, DH), F32),
            pltpu.VMEM((KB, RW), I32),
            pltpu.VMEM((KB, RW), I32),
            pltpu.VMEM((CH,), F32),
            pltpu.VMEM((CH, DH), F32),
            pltpu.SemaphoreType.DMA,
            pltpu.SemaphoreType.DMA,
        ],
    )
    def k(ta_ref, tb_ref, g_ref, s_ref, a_ref, z_ref, out_ref,
          tbl_sh, acc_sh, gi_v, si_v, ae_v, rows_v, sem, semi):
        cid = lax.axis_index("c")
        sid = lax.axis_index("s")
        t0 = sid * RT

        @pl.when(cid == 0)
        def _sa():
            pltpu.sync_copy(ta_ref.at[pl.ds(t0, RT), :],
                            tbl_sh.at[pl.ds(t0, RT), :])

        @pl.when(cid == 1)
        def _sb():
            pltpu.sync_copy(tb_ref.at[pl.ds(t0, RT), :],
                            tbl_sh.at[pl.ds(t0, RT), :])

        a0 = sid * RA

        def zc(i, carry):
            pltpu.sync_copy(z_ref.at[:, pl.ds(0, DH)],
                            acc_sh.at[pl.ds(a0 + i * 64, 64), :])
            return carry

        lax.fori_loop(0, RA // 64, zc, 0)
        plsc.subcore_barrier()

        base0 = sid * EPT
        lane = lax.iota(I32, 16)
        c0 = lane
        c1 = lane + 16
        m8 = lane < 8

        def chunk(c, carry):
            b = base0 + c * CH
            brow = pl.multiple_of(b // RW, 8)
            d1 = pltpu.async_copy(g_ref.at[pl.ds(brow, KB), :], gi_v, semi)
            d2 = pltpu.async_copy(s_ref.at[pl.ds(brow, KB), :], si_v, semi)
            d3 = pltpu.async_copy(a_ref.at[pl.ds(b, CH)], ae_v, semi)
            d1.wait()
            d2.wait()
            d3.wait()
            descs = [
                pltpu.async_copy(
                    tbl_sh.at[gi_v.at[j]],
                    rows_v.at[pl.ds(j * RW, RW), :], sem)
                for j in range(KB)
            ]
            for d in descs:
                d.wait()

            def grp16(g, cg):
                eb = g * 16
                wv = ae_v[pl.ds(eb, 16)]
                for kk in range(16):
                    e = eb + kk
                    w = wv[kk]
                    ef = jnp.full((16,), 0, I32) + e
                    v0 = plsc.load_gather(rows_v, [ef, c0])
                    plsc.store_scatter(rows_v, [ef, c0], v0 * w)
                    v1 = plsc.load_gather(rows_v, [ef, c1], mask=m8)
                    plsc.store_scatter(rows_v, [ef, c1], v1 * w, mask=m8)
                return cg

            lax.fori_loop(0, CH // 16, grp16, 0)
            for j in range(KB):
                pltpu.sync_copy(
                    rows_v.at[pl.ds(j * RW, RW), :],
                    acc_sh.at[si_v.at[j]], add=True)
            return carry

        lax.fori_loop(0, EPT // CH, chunk, 0)
        plsc.subcore_barrier()

        def oc(i, carry):
            rr = a0 + i * 64
            pltpu.sync_copy(acc_sh.at[pl.ds(rr, 64), :],
                            out_ref.at[cid, pl.ds(rr, 64), :])
            return carry

        lax.fori_loop(0, RA // 64, oc, 0)

    return k(ta, tb, gidx2, sidx2, ae_f, zrows)


def _sc_pool(h_flat, batch_p, Np, Gp):
    H = 35
    RPT = Np // NW
    GW = Gp * D

    @functools.partial(
        pl.kernel,
        out_type=[
            jax.ShapeDtypeStruct((NW, GW), F32),
            jax.ShapeDtypeStruct((NW, Gp), F32),
        ],
        mesh=_mesh(),
        compiler_params=pltpu.CompilerParams(needs_layout_passes=False),
        scratch_types=[
            pltpu.VMEM((GW,), F32),
            pltpu.VMEM((Gp,), F32),
            pltpu.VMEM((RPT * H,), F32),
            pltpu.VMEM((RPT,), I32),
        ],
    )
    def k(h_ref, b_ref, op_ref, oc_ref, acc_v, cnt_v, hr_v, bt_v):
        wid = lax.axis_index("s") * NC + lax.axis_index("c")
        _zero_1d(acc_v, GW)
        _zero_1d(cnt_v, Gp)
        row0 = wid * RPT
        pltpu.sync_copy(h_ref.at[pl.ds(row0 * H, RPT * H)], hr_v)
        pltpu.sync_copy(b_ref.at[pl.ds(row0, RPT)], bt_v)

        lane = lax.iota(I32, 16)
        m3 = lane < 3
        ones = jnp.ones((16,), F32)

        def grp16(g, carry):
            rb0 = g * 16
            bv = bt_v[pl.ds(rb0, 16)]
            plsc.addupdate_scatter(cnt_v, [bv], ones)
            for kk in range(16):
                r = rb0 + kk
                gk = bv[kk]
                rb = (jnp.full((16,), 0, I32) + r * H) + lane
                ob = (jnp.full((16,), 0, I32) + gk * D) + lane
                v0 = plsc.load_gather(hr_v, [rb])
                plsc.addupdate_scatter(acc_v, [ob], v0)
                v1 = plsc.load_gather(hr_v, [rb + 16])
                plsc.addupdate_scatter(acc_v, [ob + 16], v1)
                v2 = plsc.load_gather(hr_v, [rb + 32], mask=m3)
                plsc.addupdate_scatter(acc_v, [ob + 32], v2, mask=m3)
            return carry

        lax.fori_loop(0, RPT // 16, grp16, 0)
        pltpu.sync_copy(acc_v, op_ref.at[wid])
        pltpu.sync_copy(cnt_v, oc_ref.at[wid])

    return k(h_flat, batch_p)


def _tc_embed(x_p, W, b, n_real):
    Np, IN = x_p.shape
    H = W.shape[1]

    def body(x_ref, w_ref, b_ref, o_ref):
        i = pl.program_id(0)
        h = jnp.dot(x_ref[...], w_ref[...], preferred_element_type=F32) + b_ref[...]
        rid = lax.broadcasted_iota(I32, (BN, H), 0) + i * BN
        o_ref[...] = jnp.where(rid < n_real, h, 0.0)

    return pl.pallas_call(
        body,
        grid=(Np // BN,),
        in_specs=[
            pl.BlockSpec((BN, IN), lambda i: (i, 0)),
            pl.BlockSpec(W.shape, lambda i: (0, 0)),
            pl.BlockSpec((1, H), lambda i: (0, 0)),
        ],
        out_specs=pl.BlockSpec((BN, H), lambda i: (i, 0)),
        out_shape=jax.ShapeDtypeStruct((Np, H), F32),
    )(x_p, W, b)


def _tc_xl(h, W, attx):
    Np, H = h.shape

    def body(h_ref, w_ref, a_ref, xl_ref, sx_ref, mx_ref):
        i = pl.program_id(0)
        xl = jnp.dot(h_ref[...], w_ref[...], preferred_element_type=F32)
        xl48 = jnp.concatenate([xl, jnp.zeros((BN, 48 - H), F32)], axis=1)
        xl_ref[...] = xl48.reshape(BN, 2, 24)
        sx = jnp.dot(xl, a_ref[...], preferred_element_type=F32)
        sx_ref[...] = sx
        m = jnp.max(sx)

        @pl.when(i == 0)
        def _init():
            mx_ref[...] = jnp.full((1, 16), -3.4e38, F32)

        mx_ref[...] = jnp.maximum(mx_ref[...], m)

    return pl.pallas_call(
        body,
        grid=(Np // BN,),
        in_specs=[
            pl.BlockSpec((BN, H), lambda i: (i, 0)),
            pl.BlockSpec(W.shape, lambda i: (0, 0)),
            pl.BlockSpec((H, 1), lambda i: (0, 0)),
        ],
        out_specs=[
            pl.BlockSpec((BN, 2, 24), lambda i: (i, 0, 0)),
            pl.BlockSpec((BN, 1), lambda i: (i, 0)),
            pl.BlockSpec((1, 16), lambda i: (0, 0)),
        ],
        out_shape=[
            jax.ShapeDtypeStruct((Np, 2, 24), F32),
            jax.ShapeDtypeStruct((Np, 1), F32),
            jax.ShapeDtypeStruct((1, 16), F32),
        ],
    )(h, W, attx)


def _tc_se(attr_p, W, atte):
    Mp, H = attr_p.shape

    def body(a_ref, w_ref, at_ref, se_ref, mx_ref):
        i = pl.program_id(0)
        el = jnp.dot(a_ref[...], w_ref[...], preferred_element_type=F32)
        se = jnp.dot(el, at_ref[...], preferred_element_type=F32)
        se_ref[...] = se
        m = jnp.max(se)

        @pl.when(i == 0)
        def _init():
            mx_ref[...] = jnp.full((1, 16), -3.4e38, F32)

        mx_ref[...] = jnp.maximum(mx_ref[...], m)

    return pl.pallas_call(
        body,
        grid=(Mp // BN,),
        in_specs=[
            pl.BlockSpec((BN, H), lambda i: (i, 0)),
            pl.BlockSpec(W.shape, lambda i: (0, 0)),
            pl.BlockSpec((H, 1), lambda i: (0, 0)),
        ],
        out_specs=[
            pl.BlockSpec((BN, 1), lambda i: (i, 0)),
            pl.BlockSpec((1, 16), lambda i: (0, 0)),
        ],
        out_shape=[
            jax.ShapeDtypeStruct((Mp, 1), F32),
            jax.ShapeDtypeStruct((1, 16), F32),
        ],
    )(attr_p, W, atte)


def _tc_T(acc1p, asum_p, cnthe_p):
    Mp = acc1p.shape[1]

    def body(a1_ref, as_ref, ch_ref, t_ref):
        acc = jnp.concatenate([a1_ref[0], a1_ref[1]], axis=1)
        asum = jnp.sum(as_ref[...], axis=0)
        cnt = jnp.sum(ch_ref[...], axis=0)
        be = jnp.where(cnt > 0, 1.0 / cnt, 0.0)
        rden = 1.0 / (asum + 1e-16)
        t48 = (be * (rden * rden))[:, None] * acc
        t_ref[...] = t48.reshape(BN, 2, 24)

    return pl.pallas_call(
        body,
        grid=(Mp // BN,),
        in_specs=[
            pl.BlockSpec((NC, BN, 24), lambda i: (0, i, 0)),
            pl.BlockSpec((NW, BN), lambda i: (0, i)),
            pl.BlockSpec((NW, BN), lambda i: (0, i)),
        ],
        out_specs=pl.BlockSpec((BN, 2, 24), lambda i: (i, 0, 0)),
        out_shape=jax.ShapeDtypeStruct((Mp, 2, 24), F32),
    )(acc1p, asum_p, cnthe_p)


def _tc_F(acc2p, cntsrc_p, bias, n_real, H):
    Np = acc2p.shape[1]

    def body(a2_ref, cs_ref, b_ref, h_ref):
        i = pl.program_id(0)
        acc = jnp.concatenate([a2_ref[0], a2_ref[1]], axis=1)
        cnt = jnp.sum(cs_ref[...], axis=0)
        dv = jnp.where(cnt > 0, 1.0 / cnt, 0.0)
        h = dv[:, None] * acc[:, :H] + b_ref[...]
        rid = lax.broadcasted_iota(I32, (BN, H), 0) + i * BN
        h_ref[...] = jnp.where(rid < n_real, h, 0.0)

    return pl.pallas_call(
        body,
        grid=(Np // BN,),
        in_specs=[
            pl.BlockSpec((NC, BN, 24), lambda i: (0, i, 0)),
            pl.BlockSpec((NW, BN), lambda i: (0, i)),
            pl.BlockSpec((1, H), lambda i: (0, 0)),
        ],
        out_specs=pl.BlockSpec((BN, H), lambda i: (i, 0)),
        out_shape=jax.ShapeDtypeStruct((Np, H), F32),
    )(acc2p, cntsrc_p, bias)


def _tc_final(pp, cp, Wp_pad, bp, Wo, bo, G):
    def body(pp_ref, cp_ref, wp_ref, bp_ref, wo_ref, bo_ref, o_ref):
        pooled = jnp.sum(pp_ref[...], axis=0)
        cnt = jnp.sum(cp_ref[...], axis=0)
        pooled = pooled / jnp.maximum(cnt, 1.0)[:, None]
        z = jnp.dot(pooled, wp_ref[...], preferred_element_type=F32) + bp_ref[...]
        p = jnp.maximum(z, 0.0) + jnp.log(1.0 + jnp.exp(-jnp.abs(z)))
        o_ref[...] = jnp.dot(p, wo_ref[...], preferred_element_type=F32) + bo_ref[...]

    Gp = pp.shape[1]
    HOUT = Wp_pad.shape[1]
    return pl.pallas_call(
        body,
        in_specs=[
            pl.BlockSpec((NW, G, D), lambda: (0, 0, 0)),
            pl.BlockSpec((NW, G), lambda: (0, 0)),
            pl.BlockSpec(Wp_pad.shape, lambda: (0, 0)),
            pl.BlockSpec((1, HOUT), lambda: (0, 0)),
            pl.BlockSpec(Wo.shape, lambda: (0, 0)),
            pl.BlockSpec((1, 1), lambda: (0, 0)),
        ],
        out_specs=pl.BlockSpec((G, 1), lambda: (0, 0)),
        out_shape=jax.ShapeDtypeStruct((G, 1), F32),
    )(pp, cp, Wp_pad, bp, Wo, bo)


def kernel(x, hyperedge_index, hyperedge_attr, batch, W_embed, b_embed,
           W0, att0, bias0, W1, att1, bias1, W2, att2, bias2,
           W_proj, b_proj, W_out, b_out):
    N, IN = x.shape
    M, H = hyperedge_attr.shape
    E = hyperedge_index.shape[1]
    G = 512
    HOUT = W_proj.shape[1]

    Np = ((N + BN) // BN) * BN
    Mp = ((M + BN) // BN) * BN
    EPW_Q = NW * 1600
    Ep = ((E + EPW_Q - 1) // EPW_Q) * EPW_Q
    Gp = G + 32

    src = hyperedge_index[0]
    he = hyperedge_index[1]
    src_f = jnp.concatenate([src, jnp.full((Ep - E,), N, I32)])
    he_f = jnp.concatenate([he, jnp.full((Ep - E,), M, I32)])
    src2 = src_f.reshape(Ep // 64, 64)
    he2 = he_f.reshape(Ep // 64, 64)
    x_p = jnp.pad(x, ((0, Np - N), (0, 0)))
    attr_p = jnp.pad(hyperedge_attr, ((0, Mp - M), (0, 0)))
    batch_p = jnp.concatenate([batch, jnp.full((Np - N,), G, I32)])
    zrows = jnp.zeros((64, D), F32)

    cn_p, cm_p = _sc_degree(src_f, he_f, Np, Mp)

    h = _tc_embed(x_p, W_embed, b_embed.reshape(1, H), N)

    layers = ((W0, att0, bias0), (W1, att1, bias1), (W2, att2, bias2))
    for (W, att, bias) in layers:
        attx = att[:H].reshape(H, 1)
        atte = att[H:].reshape(H, 1)
        xlp, sx, mxs = _tc_xl(h, W, attx)
        se, mxe = _tc_se(attr_p, W, atte)
        ae_f, as_p = _sc_alpha(src_f, he_f, sx.reshape(Np), se.reshape(Mp),
                               mxs.reshape(16), mxe.reshape(16))
        acc1p = _sc_passS(xlp[:, 0, :], xlp[:, 1, :], src2, he2, ae_f,
                          Mp, zrows)
        T2 = _tc_T(acc1p, as_p, cm_p)
        acc2p = _sc_passS(T2[:, 0, :], T2[:, 1, :], he2, src2, ae_f,
                          Np, zrows)
        h = _tc_F(acc2p, cn_p, bias.reshape(1, H), N, H)

    pp_f, cp = _sc_pool(h.reshape(Np * H), batch_p, Np, Gp)
    pp = pp_f.reshape(NW, Gp, D)[:, :G, :]
    Wp_pad = jnp.pad(W_proj, ((0, D - H), (0, 0)))
    return _tc_final(pp, cp[:, :G], Wp_pad, b_proj.reshape(1, HOUT),
                     W_out, b_out.reshape(1, 1), G)
